# baseline (speedup 1.0000x reference)
import jax
import jax.numpy as jnp
from jax import lax
from jax.experimental import pallas as pl
from jax.experimental.pallas import tpu as pltpu
from jax.experimental.pallas import tpu_sc as plsc

NC = 2
NS = 16
L = 16
NW = NC * NS

NUM_ROWS = 1000000
BATCH = 16384
DIM = 32
B_PER_W = BATCH // NW
IDXC = 128
N_IDXC = B_PER_W // IDXC
GROUPS = B_PER_W // L


def _body(uid_hbm, cid_hbm, umat_hbm, imat_hbm, out_hbm,
          uids_v, cids_v, uidx_v, cidx_v, ubuf_v, ibuf_v, out_v, sem):
  wid = lax.axis_index("s") * NC + lax.axis_index("c")
  base = wid * B_PER_W

  for j in range(N_IDXC):
    pltpu.sync_copy(uid_hbm.at[pl.ds(base + j * IDXC, IDXC)], uids_v.at[j])
    pltpu.sync_copy(cid_hbm.at[pl.ds(base + j * IDXC, IDXC)], cids_v.at[j])

  def build(j, _):
    for d in range(DIM):
      off = jnp.full((L,), d * NUM_ROWS, jnp.int32)
      for k in range(IDXC // L):
        s = pl.ds(k * L, L)
        uidx_v[j, d, s] = uids_v[j, s] + off
        cidx_v[j, d, s] = cids_v[j, s] + off
    return _

  lax.fori_loop(0, N_IDXC, build, 0, unroll=False)

  copies = []
  for j in range(N_IDXC):
    s = pl.ds(j * IDXC, IDXC)
    for d in range(DIM):
      copies.append(pltpu.async_copy(
          umat_hbm.at[uidx_v.at[j, d]], ubuf_v.at[d, s], sem))
      copies.append(pltpu.async_copy(
          imat_hbm.at[cidx_v.at[j, d]], ibuf_v.at[d, s], sem))
  for c in copies:
    c.wait()

  def compute(g, _):
    s = pl.ds(pl.multiple_of(g * L, L), L)
    acc = jnp.zeros((L,), jnp.float32)
    for d in range(DIM):
      acc = acc + ubuf_v[d, s] * ibuf_v[d, s]
    e = jnp.exp(-jnp.abs(acc))
    sig = jnp.where(acc >= 0, 1.0 / (1.0 + e), e / (1.0 + e))
    out_v[s] = sig
    return _

  lax.fori_loop(0, GROUPS, compute, 0, unroll=False)

  pltpu.sync_copy(out_v, out_hbm.at[pl.ds(base, B_PER_W)])


@jax.jit
def kernel(user_ids, content_ids, user_matrix, item_matrix):
  uid = user_ids.astype(jnp.int32)
  cid = content_ids.astype(jnp.int32)
  umat = user_matrix.T.reshape(DIM * NUM_ROWS)
  imat = item_matrix.T.reshape(DIM * NUM_ROWS)

  mesh = plsc.VectorSubcoreMesh(
      core_axis_name="c", subcore_axis_name="s", num_cores=NC,
      num_subcores=NS)

  run = pl.kernel(
      _body,
      out_type=jax.ShapeDtypeStruct((BATCH,), jnp.float32),
      mesh=mesh,
      compiler_params=pltpu.CompilerParams(needs_layout_passes=False),
      scratch_types=[
          pltpu.VMEM((N_IDXC, IDXC), jnp.int32),
          pltpu.VMEM((N_IDXC, IDXC), jnp.int32),
          pltpu.VMEM((N_IDXC, DIM, IDXC), jnp.int32),
          pltpu.VMEM((N_IDXC, DIM, IDXC), jnp.int32),
          pltpu.VMEM((DIM, B_PER_W), jnp.float32),
          pltpu.VMEM((DIM, B_PER_W), jnp.float32),
          pltpu.VMEM((B_PER_W,), jnp.float32),
          pltpu.SemaphoreType.DMA,
      ],
  )
  return run(uid, cid, umat, imat)


# speedup vs baseline: 4.4127x; 4.4127x over previous
import jax
import jax.numpy as jnp
from jax import lax
from jax.experimental import pallas as pl
from jax.experimental.pallas import tpu as pltpu
from jax.experimental.pallas import tpu_sc as plsc

NC = 2
NS = 16
L = 16
NW = NC * NS

NUM_ROWS = 1000000
NUM_PAD = 1000064
BATCH = 16384
DIM = 32
SUB = 8
NBLK = DIM // SUB
TILES = NUM_PAD // 128
TABLE_TILES = NBLK * TILES
KMAX = -(-TABLE_TILES // NW)
OUT_WORDS = NBLK * NUM_PAD * SUB

B_PER_W = BATCH // NW
IDXC = 128
N_IDXC = B_PER_W // IDXC
GROUPS = B_PER_W // L


def _p1_body(umat_hbm, imat_hbm, uout_hbm, iout_hbm,
             tvin_v, tvout_v, rsem, wsem):
  wid = lax.axis_index("s") * NC + lax.axis_index("c")
  lam = lax.iota(jnp.int32, L)
  perm = lax.shift_left(lam & jnp.full((L,), 7, jnp.int32), 7) + \
      lax.shift_right_logical(lam, 3)

  for src, dst in ((umat_hbm, uout_hbm), (imat_hbm, iout_hbm)):
    def tile_of(k):
      t = (wid + k * NW) % TABLE_TILES
      b = t // TILES
      return b, (t % TILES) * 128

    def start_read(k, slot):
      b, c128 = tile_of(k)
      pltpu.async_copy(src.at[b, slice(None), pl.ds(c128, 128)],
                       tvin_v.at[slot], rsem)

    def wait_one(ref, sem):
      pltpu.make_async_copy(dst.at[pl.ds(0, 1024)], ref, sem).wait()

    start_read(0, 0)

    def step(k, carry):
      start_read((k + 1) % KMAX, (k + 1) % 2)
      wait_one(tvout_v.at[0], rsem)
      buf = k % 2
      bufv = jnp.broadcast_to(buf, (L,))
      sv = lam & jnp.full((L,), 7, jnp.int32)
      for p in range(64):
        lv = jnp.full((L,), 2 * p, jnp.int32) + lax.shift_right_logical(lam, 3)
        val = plsc.load_gather(tvin_v, [bufv, sv, lv])
        tvout_v[buf, pl.ds(pl.multiple_of(2 * p * SUB, L), L)] = val
      b, c128 = tile_of(k)
      base = b * (NUM_PAD * SUB) + c128 * SUB
      pltpu.async_copy(tvout_v.at[buf], dst.at[pl.ds(base, 1024)], wsem)

      @pl.when(k >= 1)
      def _drain_one():
        wait_one(tvout_v.at[0], wsem)
      return carry

    lax.fori_loop(0, KMAX, step, 0, unroll=False)
    wait_one(tvout_v.at[0], rsem)
    wait_one(tvout_v.at[0], wsem)


def _p2_body(uid_hbm, cid_hbm, ubm_hbm, ibm_hbm, out_hbm,
             uids_v, cids_v, uidx_v, cidx_v, ubuf_v, ibuf_v, out_v, sem):
  wid = lax.axis_index("s") * NC + lax.axis_index("c")
  base = wid * B_PER_W

  for j in range(N_IDXC):
    pltpu.sync_copy(uid_hbm.at[pl.ds(base + j * IDXC, IDXC)], uids_v.at[j])
    pltpu.sync_copy(cid_hbm.at[pl.ds(base + j * IDXC, IDXC)], cids_v.at[j])

  for j in range(N_IDXC):
    for b in range(NBLK):
      off = jnp.full((L,), b * NUM_PAD, jnp.int32)
      for k in range(IDXC // L):
        s = pl.ds(k * L, L)
        uidx_v[j, b, s] = uids_v[j, s] + off
        cidx_v[j, b, s] = cids_v[j, s] + off

  copies = []
  for j in range(N_IDXC):
    for b in range(NBLK):
      copies.append(pltpu.async_copy(
          ubm_hbm.at[uidx_v.at[j, b]], ubuf_v.at[j, b], sem))
      copies.append(pltpu.async_copy(
          ibm_hbm.at[cidx_v.at[j, b]], ibuf_v.at[j, b], sem))
  for c in copies:
    c.wait()

  lam = lax.iota(jnp.int32, L)

  def compute(g, _):
    j = g // SUB
    jv = jnp.broadcast_to(j, (L,))
    pv = jnp.broadcast_to((g & 7) * L, (L,)) + lam
    acc = jnp.zeros((L,), jnp.float32)
    for b in range(NBLK):
      bv = jnp.full((L,), b, jnp.int32)
      for s in range(SUB):
        sv = jnp.full((L,), s, jnp.int32)
        u = plsc.load_gather(ubuf_v, [jv, bv, pv, sv])
        v = plsc.load_gather(ibuf_v, [jv, bv, pv, sv])
        acc = acc + u * v
    e = jnp.exp(-jnp.abs(acc))
    sig = jnp.where(acc >= 0, 1.0 / (1.0 + e), e / (1.0 + e))
    out_v[pl.ds(g * L, L)] = sig
    return _

  lax.fori_loop(0, GROUPS, compute, 0, unroll=False)

  pltpu.sync_copy(out_v, out_hbm.at[pl.ds(base, B_PER_W)])


@jax.jit
def kernel(user_ids, content_ids, user_matrix, item_matrix):
  uid = user_ids.astype(jnp.int32)
  cid = content_ids.astype(jnp.int32)
  umat = user_matrix.T.reshape(NBLK, SUB, NUM_ROWS)
  imat = item_matrix.T.reshape(NBLK, SUB, NUM_ROWS)

  mesh = plsc.VectorSubcoreMesh(
      core_axis_name="c", subcore_axis_name="s", num_cores=NC,
      num_subcores=NS)

  p1 = pl.kernel(
      _p1_body,
      out_type=(jax.ShapeDtypeStruct((OUT_WORDS,), jnp.float32),
                jax.ShapeDtypeStruct((OUT_WORDS,), jnp.float32)),
      mesh=mesh,
      compiler_params=pltpu.CompilerParams(
          needs_layout_passes=False, use_tc_tiling_on_sc=True,
          disable_bounds_checks=True),
      scratch_types=[
          pltpu.VMEM((2, SUB, 128), jnp.float32),
          pltpu.VMEM((2, 1024), jnp.float32),
          pltpu.SemaphoreType.DMA,
          pltpu.SemaphoreType.DMA,
      ],
  )
  ubm, ibm = p1(umat, imat)

  p2 = pl.kernel(
      _p2_body,
      out_type=jax.ShapeDtypeStruct((BATCH,), jnp.float32),
      mesh=mesh,
      compiler_params=pltpu.CompilerParams(
          needs_layout_passes=False, use_tc_tiling_on_sc=False,
          disable_bounds_checks=True),
      scratch_types=[
          pltpu.VMEM((N_IDXC, IDXC), jnp.int32),
          pltpu.VMEM((N_IDXC, IDXC), jnp.int32),
          pltpu.VMEM((N_IDXC, NBLK, IDXC), jnp.int32),
          pltpu.VMEM((N_IDXC, NBLK, IDXC), jnp.int32),
          pltpu.VMEM((N_IDXC, NBLK, IDXC, SUB), jnp.float32),
          pltpu.VMEM((N_IDXC, NBLK, IDXC, SUB), jnp.float32),
          pltpu.VMEM((B_PER_W,), jnp.float32),
          pltpu.SemaphoreType.DMA,
      ],
  )
  return p2(uid, cid, ubm.reshape(NBLK * NUM_PAD, SUB),
            ibm.reshape(NBLK * NUM_PAD, SUB))


# speedup vs baseline: 5.0037x; 1.1339x over previous
import jax
import jax.numpy as jnp
from jax import lax
from jax.experimental import pallas as pl
from jax.experimental.pallas import tpu as pltpu
from jax.experimental.pallas import tpu_sc as plsc

NC = 2
NS = 16
L = 16
NW = NC * NS

NUM_ROWS = 1000000
NUM_PAD = 1000064
BATCH = 16384
DIM = 32
SUB = 8
NBLK = DIM // SUB
TILES = NUM_PAD // 128
TABLE_TILES = NBLK * TILES
KMAX = -(-TABLE_TILES // NW)
OUT_WORDS = NBLK * NUM_PAD * SUB

B_PER_W = BATCH // NW
IDXC = 128
N_IDXC = B_PER_W // IDXC
GROUPS = B_PER_W // L


def _p1_body(umat_hbm, imat_hbm, uout_hbm, iout_hbm,
             tvin_v, tvout_v, rsem, wsem):
  wid = lax.axis_index("s") * NC + lax.axis_index("c")
  lam = lax.iota(jnp.int32, L)
  perm = lax.shift_left(lam & jnp.full((L,), 7, jnp.int32), 7) + \
      lax.shift_right_logical(lam, 3)

  for src, dst in ((umat_hbm, uout_hbm), (imat_hbm, iout_hbm)):
    def tile_of(k):
      t = (wid + k * NW) % TABLE_TILES
      b = t // TILES
      return b, (t % TILES) * 128

    def start_read(k, slot):
      b, c128 = tile_of(k)
      pltpu.async_copy(src.at[b, slice(None), pl.ds(c128, 128)],
                       tvin_v.at[slot], rsem)

    def wait_one(ref, sem):
      pltpu.make_async_copy(dst.at[pl.ds(0, 1024)], ref, sem).wait()

    for kk in range(3):
      start_read(kk, kk)

    def step(k, carry):
      start_read((k + 3) % KMAX, (k + 3) % 4)
      wait_one(tvout_v.at[0], rsem)
      buf = k % 4
      bufv = jnp.broadcast_to(buf, (L,))
      sv = lam & jnp.full((L,), 7, jnp.int32)
      for p in range(64):
        lv = jnp.full((L,), 2 * p, jnp.int32) + lax.shift_right_logical(lam, 3)
        val = plsc.load_gather(tvin_v, [bufv, sv, lv])
        tvout_v[buf, pl.ds(pl.multiple_of(2 * p * SUB, L), L)] = val
      b, c128 = tile_of(k)
      base = b * (NUM_PAD * SUB) + c128 * SUB
      pltpu.async_copy(tvout_v.at[buf], dst.at[pl.ds(base, 1024)], wsem)

      @pl.when(k >= 3)
      def _drain_one():
        wait_one(tvout_v.at[0], wsem)
      return carry

    lax.fori_loop(0, KMAX, step, 0, unroll=False)
    for _ in range(3):
      wait_one(tvout_v.at[0], rsem)
      wait_one(tvout_v.at[0], wsem)


def _p2_body(uid_hbm, cid_hbm, ubm_hbm, ibm_hbm, out_hbm,
             uids_v, cids_v, uidx_v, cidx_v, ubuf_v, ibuf_v, out_v, sem):
  wid = lax.axis_index("s") * NC + lax.axis_index("c")
  base = wid * B_PER_W

  for j in range(N_IDXC):
    pltpu.sync_copy(uid_hbm.at[pl.ds(base + j * IDXC, IDXC)], uids_v.at[j])
    pltpu.sync_copy(cid_hbm.at[pl.ds(base + j * IDXC, IDXC)], cids_v.at[j])

  for j in range(N_IDXC):
    for b in range(NBLK):
      off = jnp.full((L,), b * NUM_PAD, jnp.int32)
      for k in range(IDXC // L):
        s = pl.ds(k * L, L)
        uidx_v[j, b, s] = uids_v[j, s] + off
        cidx_v[j, b, s] = cids_v[j, s] + off

  copies = []
  for j in range(N_IDXC):
    for b in range(NBLK):
      copies.append(pltpu.async_copy(
          ubm_hbm.at[uidx_v.at[j, b]], ubuf_v.at[j, b], sem))
      copies.append(pltpu.async_copy(
          ibm_hbm.at[cidx_v.at[j, b]], ibuf_v.at[j, b], sem))
  for c in copies:
    c.wait()

  lam = lax.iota(jnp.int32, L)

  def compute(g, _):
    j = g // SUB
    jv = jnp.broadcast_to(j, (L,))
    pv = jnp.broadcast_to((g & 7) * L, (L,)) + lam
    acc = jnp.zeros((L,), jnp.float32)
    for b in range(NBLK):
      bv = jnp.full((L,), b, jnp.int32)
      for s in range(SUB):
        sv = jnp.full((L,), s, jnp.int32)
        u = plsc.load_gather(ubuf_v, [jv, bv, pv, sv])
        v = plsc.load_gather(ibuf_v, [jv, bv, pv, sv])
        acc = acc + u * v
    e = jnp.exp(-jnp.abs(acc))
    sig = jnp.where(acc >= 0, 1.0 / (1.0 + e), e / (1.0 + e))
    out_v[pl.ds(g * L, L)] = sig
    return _

  lax.fori_loop(0, GROUPS, compute, 0, unroll=False)

  pltpu.sync_copy(out_v, out_hbm.at[pl.ds(base, B_PER_W)])


@jax.jit
def kernel(user_ids, content_ids, user_matrix, item_matrix):
  uid = user_ids.astype(jnp.int32)
  cid = content_ids.astype(jnp.int32)
  umat = user_matrix.T.reshape(NBLK, SUB, NUM_ROWS)
  imat = item_matrix.T.reshape(NBLK, SUB, NUM_ROWS)

  mesh = plsc.VectorSubcoreMesh(
      core_axis_name="c", subcore_axis_name="s", num_cores=NC,
      num_subcores=NS)

  p1 = pl.kernel(
      _p1_body,
      out_type=(jax.ShapeDtypeStruct((OUT_WORDS,), jnp.float32),
                jax.ShapeDtypeStruct((OUT_WORDS,), jnp.float32)),
      mesh=mesh,
      compiler_params=pltpu.CompilerParams(
          needs_layout_passes=False, use_tc_tiling_on_sc=True,
          disable_bounds_checks=True),
      scratch_types=[
          pltpu.VMEM((4, SUB, 128), jnp.float32),
          pltpu.VMEM((4, 1024), jnp.float32),
          pltpu.SemaphoreType.DMA,
          pltpu.SemaphoreType.DMA,
      ],
  )
  ubm, ibm = p1(umat, imat)

  p2 = pl.kernel(
      _p2_body,
      out_type=jax.ShapeDtypeStruct((BATCH,), jnp.float32),
      mesh=mesh,
      compiler_params=pltpu.CompilerParams(
          needs_layout_passes=False, use_tc_tiling_on_sc=False,
          disable_bounds_checks=True),
      scratch_types=[
          pltpu.VMEM((N_IDXC, IDXC), jnp.int32),
          pltpu.VMEM((N_IDXC, IDXC), jnp.int32),
          pltpu.VMEM((N_IDXC, NBLK, IDXC), jnp.int32),
          pltpu.VMEM((N_IDXC, NBLK, IDXC), jnp.int32),
          pltpu.VMEM((N_IDXC, NBLK, IDXC, SUB), jnp.float32),
          pltpu.VMEM((N_IDXC, NBLK, IDXC, SUB), jnp.float32),
          pltpu.VMEM((B_PER_W,), jnp.float32),
          pltpu.SemaphoreType.DMA,
      ],
  )
  return p2(uid, cid, ubm.reshape(NBLK * NUM_PAD, SUB),
            ibm.reshape(NBLK * NUM_PAD, SUB))


# speedup vs baseline: 11.1208x; 2.2225x over previous
import jax
import jax.numpy as jnp
from jax import lax
from jax.experimental import pallas as pl
from jax.experimental.pallas import tpu as pltpu
from jax.experimental.pallas import tpu_sc as plsc

NC = 2
NS = 16
L = 16
NW = NC * NS

NUM_ROWS = 1000000
BATCH = 16384
DIM = 32
SUB = 8
NBLK = DIM // SUB
TILES = -(-NUM_ROWS // 128)
TABLE_TILES = NBLK * TILES
KMAX = -(-TABLE_TILES // NW)
BLOCK_WORDS = TILES * 1024

B_PER_W = BATCH // NW
IDXC = 128
N_IDXC = B_PER_W // IDXC
GROUPS = B_PER_W // L
NBUF = 4


def _p1_body(umat_hbm, imat_hbm, uout_hbm, iout_hbm, tv_v, rsem, wsem):
  wid = lax.axis_index("s") * NC + lax.axis_index("c")

  for src, dst in ((umat_hbm, uout_hbm), (imat_hbm, iout_hbm)):
    def tile_of(k):
      t = (wid + k * NW) % TABLE_TILES
      return t // TILES, t % TILES

    def start_read(k, slot):
      b, c = tile_of(k)
      pltpu.async_copy(src.at[b, slice(None), pl.ds(c * 128, 128)],
                       tv_v.at[slot], rsem)

    def wait_one(sem):
      pltpu.make_async_copy(dst.at[0], tv_v.at[0], sem).wait()

    for kk in range(NBUF - 1):
      start_read(kk, kk)

    def step(k, carry):
      start_read((k + NBUF - 1) % KMAX, (k + NBUF - 1) % NBUF)
      wait_one(rsem)
      b, c = tile_of(k)
      pltpu.async_copy(tv_v.at[k % NBUF], dst.at[b * TILES + c], wsem)

      @pl.when(k >= NBUF - 1)
      def _drain_one():
        wait_one(wsem)
      return carry

    lax.fori_loop(0, KMAX, step, 0, unroll=False)
    for _ in range(NBUF - 1):
      wait_one(rsem)
      wait_one(wsem)


def _p2_body(uid_hbm, cid_hbm, uflat_hbm, iflat_hbm, out_hbm,
             uids_v, cids_v, uidx_v, cidx_v, ubuf_v, ibuf_v, out_v, sem):
  wid = lax.axis_index("s") * NC + lax.axis_index("c")
  base = wid * B_PER_W

  for j in range(N_IDXC):
    pltpu.sync_copy(uid_hbm.at[pl.ds(base + j * IDXC, IDXC)], uids_v.at[j])
    pltpu.sync_copy(cid_hbm.at[pl.ds(base + j * IDXC, IDXC)], cids_v.at[j])

  def build(j, carry):
    for k in range(IDXC // L):
      s = pl.ds(k * L, L)
      for ids_v, idx_v in ((uids_v, uidx_v), (cids_v, cidx_v)):
        idv = ids_v[j, s]
        bvec = lax.shift_left(lax.shift_right_logical(idv, 7), 10) + \
            (idv & jnp.full((L,), 127, jnp.int32))
        for d in range(DIM):
          off = (d // SUB) * BLOCK_WORDS + (d % SUB) * 128
          idx_v[j, d, s] = bvec + jnp.full((L,), off, jnp.int32)
    return carry

  lax.fori_loop(0, N_IDXC, build, 0, unroll=False)

  copies = []
  for j in range(N_IDXC):
    s = pl.ds(j * IDXC, IDXC)
    for d in range(DIM):
      copies.append(pltpu.async_copy(
          uflat_hbm.at[uidx_v.at[j, d]], ubuf_v.at[d, s], sem))
      copies.append(pltpu.async_copy(
          iflat_hbm.at[cidx_v.at[j, d]], ibuf_v.at[d, s], sem))
  for c in copies:
    c.wait()

  def compute(g, carry):
    s = pl.ds(pl.multiple_of(g * L, L), L)
    acc = jnp.zeros((L,), jnp.float32)
    for d in range(DIM):
      acc = acc + ubuf_v[d, s] * ibuf_v[d, s]
    e = jnp.exp(-jnp.abs(acc))
    sig = jnp.where(acc >= 0, 1.0 / (1.0 + e), e / (1.0 + e))
    out_v[s] = sig
    return carry

  lax.fori_loop(0, GROUPS, compute, 0, unroll=False)

  pltpu.sync_copy(out_v, out_hbm.at[pl.ds(base, B_PER_W)])


@jax.jit
def kernel(user_ids, content_ids, user_matrix, item_matrix):
  uid = user_ids.astype(jnp.int32)
  cid = content_ids.astype(jnp.int32)
  umat = user_matrix.T.reshape(NBLK, SUB, NUM_ROWS)
  imat = item_matrix.T.reshape(NBLK, SUB, NUM_ROWS)

  mesh = plsc.VectorSubcoreMesh(
      core_axis_name="c", subcore_axis_name="s", num_cores=NC,
      num_subcores=NS)

  p1 = pl.kernel(
      _p1_body,
      out_type=(jax.ShapeDtypeStruct((TABLE_TILES, SUB, 128), jnp.float32),
                jax.ShapeDtypeStruct((TABLE_TILES, SUB, 128), jnp.float32)),
      mesh=mesh,
      compiler_params=pltpu.CompilerParams(
          needs_layout_passes=False, use_tc_tiling_on_sc=True,
          disable_bounds_checks=True),
      scratch_types=[
          pltpu.VMEM((NBUF, SUB, 128), jnp.float32),
          pltpu.SemaphoreType.DMA,
          pltpu.SemaphoreType.DMA,
      ],
  )
  ubm, ibm = p1(umat, imat)

  p2 = pl.kernel(
      _p2_body,
      out_type=jax.ShapeDtypeStruct((BATCH,), jnp.float32),
      mesh=mesh,
      compiler_params=pltpu.CompilerParams(
          needs_layout_passes=False, use_tc_tiling_on_sc=False,
          disable_bounds_checks=True),
      scratch_types=[
          pltpu.VMEM((N_IDXC, IDXC), jnp.int32),
          pltpu.VMEM((N_IDXC, IDXC), jnp.int32),
          pltpu.VMEM((N_IDXC, DIM, IDXC), jnp.int32),
          pltpu.VMEM((N_IDXC, DIM, IDXC), jnp.int32),
          pltpu.VMEM((DIM, B_PER_W), jnp.float32),
          pltpu.VMEM((DIM, B_PER_W), jnp.float32),
          pltpu.VMEM((B_PER_W,), jnp.float32),
          pltpu.SemaphoreType.DMA,
      ],
  )
  return p2(uid, cid, ubm.reshape(-1), ibm.reshape(-1))


# speedup vs baseline: 16.8092x; 1.5115x over previous
import jax
import jax.numpy as jnp
from jax import lax
from jax.experimental import pallas as pl
from jax.experimental.pallas import tpu as pltpu
from jax.experimental.pallas import tpu_sc as plsc

NC = 2
NS = 16
L = 16
NW = NC * NS

NUM_ROWS = 1000000
BATCH = 16384
DIM = 32
SUB = 8
NBLK = DIM // SUB
TILES = -(-NUM_ROWS // 128)
TABLE_TILES = NBLK * TILES
KMAX = -(-TABLE_TILES // NW)
BLOCK_WORDS = TILES * 1024

B_PER_W = BATCH // NW
IDXC = 128
N_IDXC = B_PER_W // IDXC
GROUPS = B_PER_W // L
NBUF = 8


def _p1_body(umat_hbm, imat_hbm, uout_hbm, iout_hbm, tv_v, rsem, wsem):
  wid = lax.axis_index("s") * NC + lax.axis_index("c")

  for src, dst in ((umat_hbm, uout_hbm), (imat_hbm, iout_hbm)):
    def tile_of(k):
      t = (wid + k * NW) % TABLE_TILES
      return t // TILES, t % TILES

    def start_read(k, slot):
      b, c = tile_of(k)
      pltpu.async_copy(src.at[b, slice(None), pl.ds(c * 128, 128)],
                       tv_v.at[slot], rsem)

    def wait_one(sem):
      pltpu.make_async_copy(dst.at[0], tv_v.at[0], sem).wait()

    for kk in range(NBUF - 1):
      start_read(kk, kk)

    def step(k, carry):
      start_read((k + NBUF - 1) % KMAX, (k + NBUF - 1) % NBUF)
      wait_one(rsem)
      b, c = tile_of(k)
      pltpu.async_copy(tv_v.at[k % NBUF], dst.at[b * TILES + c], wsem)

      @pl.when(k >= NBUF - 1)
      def _drain_one():
        wait_one(wsem)
      return carry

    lax.fori_loop(0, KMAX, step, 0, unroll=False)
    for _ in range(NBUF - 1):
      wait_one(rsem)
      wait_one(wsem)


def _p2_body(uid_hbm, cid_hbm, uflat_hbm, iflat_hbm, out_hbm,
             uids_v, cids_v, uidx_v, cidx_v, ubuf_v, ibuf_v, out_v, sem):
  wid = lax.axis_index("s") * NC + lax.axis_index("c")
  base = wid * B_PER_W

  for j in range(N_IDXC):
    pltpu.sync_copy(uid_hbm.at[pl.ds(base + j * IDXC, IDXC)], uids_v.at[j])
    pltpu.sync_copy(cid_hbm.at[pl.ds(base + j * IDXC, IDXC)], cids_v.at[j])

  def build(j, carry):
    for k in range(IDXC // L):
      s = pl.ds(k * L, L)
      for ids_v, idx_v in ((uids_v, uidx_v), (cids_v, cidx_v)):
        idv = ids_v[j, s]
        bvec = lax.shift_left(lax.shift_right_logical(idv, 7), 10) + \
            (idv & jnp.full((L,), 127, jnp.int32))
        for d in range(DIM):
          off = (d // SUB) * BLOCK_WORDS + (d % SUB) * 128
          idx_v[j, d, s] = bvec + jnp.full((L,), off, jnp.int32)
    return carry

  lax.fori_loop(0, N_IDXC, build, 0, unroll=False)

  copies = []
  for j in range(N_IDXC):
    s = pl.ds(j * IDXC, IDXC)
    for d in range(DIM):
      copies.append(pltpu.async_copy(
          uflat_hbm.at[uidx_v.at[j, d]], ubuf_v.at[d, s], sem))
      copies.append(pltpu.async_copy(
          iflat_hbm.at[cidx_v.at[j, d]], ibuf_v.at[d, s], sem))
  for c in copies:
    c.wait()

  def compute(g, carry):
    s = pl.ds(pl.multiple_of(g * L, L), L)
    acc = jnp.zeros((L,), jnp.float32)
    for d in range(DIM):
      acc = acc + ubuf_v[d, s] * ibuf_v[d, s]
    e = jnp.exp(-jnp.abs(acc))
    sig = jnp.where(acc >= 0, 1.0 / (1.0 + e), e / (1.0 + e))
    out_v[s] = sig
    return carry

  lax.fori_loop(0, GROUPS, compute, 0, unroll=False)

  pltpu.sync_copy(out_v, out_hbm.at[pl.ds(base, B_PER_W)])


@jax.jit
def kernel(user_ids, content_ids, user_matrix, item_matrix):
  uid = user_ids.astype(jnp.int32)
  cid = content_ids.astype(jnp.int32)
  umat = user_matrix.T.reshape(NBLK, SUB, NUM_ROWS)
  imat = item_matrix.T.reshape(NBLK, SUB, NUM_ROWS)

  mesh = plsc.VectorSubcoreMesh(
      core_axis_name="c", subcore_axis_name="s", num_cores=NC,
      num_subcores=NS)

  p1 = pl.kernel(
      _p1_body,
      out_type=(jax.ShapeDtypeStruct((TABLE_TILES, SUB, 128), jnp.float32),
                jax.ShapeDtypeStruct((TABLE_TILES, SUB, 128), jnp.float32)),
      mesh=mesh,
      compiler_params=pltpu.CompilerParams(
          needs_layout_passes=False, use_tc_tiling_on_sc=True,
          disable_bounds_checks=True),
      scratch_types=[
          pltpu.VMEM((NBUF, SUB, 128), jnp.float32),
          pltpu.SemaphoreType.DMA,
          pltpu.SemaphoreType.DMA,
      ],
  )
  ubm, ibm = p1(umat, imat)

  p2 = pl.kernel(
      _p2_body,
      out_type=jax.ShapeDtypeStruct((BATCH,), jnp.float32),
      mesh=mesh,
      compiler_params=pltpu.CompilerParams(
          needs_layout_passes=False, use_tc_tiling_on_sc=False,
          disable_bounds_checks=True),
      scratch_types=[
          pltpu.VMEM((N_IDXC, IDXC), jnp.int32),
          pltpu.VMEM((N_IDXC, IDXC), jnp.int32),
          pltpu.VMEM((N_IDXC, DIM, IDXC), jnp.int32),
          pltpu.VMEM((N_IDXC, DIM, IDXC), jnp.int32),
          pltpu.VMEM((DIM, B_PER_W), jnp.float32),
          pltpu.VMEM((DIM, B_PER_W), jnp.float32),
          pltpu.VMEM((B_PER_W,), jnp.float32),
          pltpu.SemaphoreType.DMA,
      ],
  )
  return p2(uid, cid, ubm.reshape(-1), ibm.reshape(-1))


# speedup vs baseline: 20.6572x; 1.2289x over previous
import jax
import jax.numpy as jnp
from jax import lax
from jax.experimental import pallas as pl
from jax.experimental.pallas import tpu as pltpu
from jax.experimental.pallas import tpu_sc as plsc

NC = 2
NS = 16
L = 16
NW = NC * NS

NUM_ROWS = 1000000
BATCH = 16384
DIM = 32
SUB = 8
NBLK = DIM // SUB
TILES = -(-NUM_ROWS // 128)
TABLE_TILES = NBLK * TILES
KMAX = -(-TABLE_TILES // NW)
BLOCK_WORDS = TILES * 1024

B_PER_W = BATCH // NW
IDXC = 128
N_IDXC = B_PER_W // IDXC
GROUPS = B_PER_W // L
NBUF = 16


def _p1_body(umat_hbm, imat_hbm, uout_hbm, iout_hbm, tv_v, rsem, wsem):
  wid = lax.axis_index("s") * NC + lax.axis_index("c")

  for src, dst in ((umat_hbm, uout_hbm), (imat_hbm, iout_hbm)):
    def tile_of(k):
      t = (wid + k * NW) % TABLE_TILES
      return t // TILES, t % TILES

    def start_read(k, slot):
      b, c = tile_of(k)
      pltpu.async_copy(src.at[b, slice(None), pl.ds(c * 128, 128)],
                       tv_v.at[slot], rsem)

    def wait_one(sem):
      pltpu.make_async_copy(dst.at[0], tv_v.at[0], sem).wait()

    for kk in range(NBUF - 1):
      start_read(kk, kk)

    def step(k, carry):
      start_read((k + NBUF - 1) % KMAX, (k + NBUF - 1) % NBUF)
      wait_one(rsem)
      b, c = tile_of(k)
      pltpu.async_copy(tv_v.at[k % NBUF], dst.at[b * TILES + c], wsem)

      @pl.when(k >= NBUF - 1)
      def _drain_one():
        wait_one(wsem)
      return carry

    lax.fori_loop(0, KMAX, step, 0, unroll=False)
    for _ in range(NBUF - 1):
      wait_one(rsem)
      wait_one(wsem)


def _p2_body(uid_hbm, cid_hbm, uflat_hbm, iflat_hbm, out_hbm,
             uids_v, cids_v, uidx_v, cidx_v, ubuf_v, ibuf_v, out_v, sem):
  wid = lax.axis_index("s") * NC + lax.axis_index("c")
  base = wid * B_PER_W

  for j in range(N_IDXC):
    pltpu.sync_copy(uid_hbm.at[pl.ds(base + j * IDXC, IDXC)], uids_v.at[j])
    pltpu.sync_copy(cid_hbm.at[pl.ds(base + j * IDXC, IDXC)], cids_v.at[j])

  def build(j, carry):
    for k in range(IDXC // L):
      s = pl.ds(k * L, L)
      for ids_v, idx_v in ((uids_v, uidx_v), (cids_v, cidx_v)):
        idv = ids_v[j, s]
        bvec = lax.shift_left(lax.shift_right_logical(idv, 7), 10) + \
            (idv & jnp.full((L,), 127, jnp.int32))
        for d in range(DIM):
          off = (d // SUB) * BLOCK_WORDS + (d % SUB) * 128
          idx_v[j, d, s] = bvec + jnp.full((L,), off, jnp.int32)
    return carry

  lax.fori_loop(0, N_IDXC, build, 0, unroll=False)

  copies = []
  for j in range(N_IDXC):
    s = pl.ds(j * IDXC, IDXC)
    for d in range(DIM):
      copies.append(pltpu.async_copy(
          uflat_hbm.at[uidx_v.at[j, d]], ubuf_v.at[d, s], sem))
      copies.append(pltpu.async_copy(
          iflat_hbm.at[cidx_v.at[j, d]], ibuf_v.at[d, s], sem))
  for c in copies:
    c.wait()

  def compute(g, carry):
    s = pl.ds(pl.multiple_of(g * L, L), L)
    acc = jnp.zeros((L,), jnp.float32)
    for d in range(DIM):
      acc = acc + ubuf_v[d, s] * ibuf_v[d, s]
    e = jnp.exp(-jnp.abs(acc))
    sig = jnp.where(acc >= 0, 1.0 / (1.0 + e), e / (1.0 + e))
    out_v[s] = sig
    return carry

  lax.fori_loop(0, GROUPS, compute, 0, unroll=False)

  pltpu.sync_copy(out_v, out_hbm.at[pl.ds(base, B_PER_W)])


@jax.jit
def kernel(user_ids, content_ids, user_matrix, item_matrix):
  uid = user_ids.astype(jnp.int32)
  cid = content_ids.astype(jnp.int32)
  umat = user_matrix.T.reshape(NBLK, SUB, NUM_ROWS)
  imat = item_matrix.T.reshape(NBLK, SUB, NUM_ROWS)

  mesh = plsc.VectorSubcoreMesh(
      core_axis_name="c", subcore_axis_name="s", num_cores=NC,
      num_subcores=NS)

  p1 = pl.kernel(
      _p1_body,
      out_type=(jax.ShapeDtypeStruct((TABLE_TILES, SUB, 128), jnp.float32),
                jax.ShapeDtypeStruct((TABLE_TILES, SUB, 128), jnp.float32)),
      mesh=mesh,
      compiler_params=pltpu.CompilerParams(
          needs_layout_passes=False, use_tc_tiling_on_sc=True,
          disable_bounds_checks=True),
      scratch_types=[
          pltpu.VMEM((NBUF, SUB, 128), jnp.float32),
          pltpu.SemaphoreType.DMA,
          pltpu.SemaphoreType.DMA,
      ],
  )
  ubm, ibm = p1(umat, imat)

  p2 = pl.kernel(
      _p2_body,
      out_type=jax.ShapeDtypeStruct((BATCH,), jnp.float32),
      mesh=mesh,
      compiler_params=pltpu.CompilerParams(
          needs_layout_passes=False, use_tc_tiling_on_sc=False,
          disable_bounds_checks=True),
      scratch_types=[
          pltpu.VMEM((N_IDXC, IDXC), jnp.int32),
          pltpu.VMEM((N_IDXC, IDXC), jnp.int32),
          pltpu.VMEM((N_IDXC, DIM, IDXC), jnp.int32),
          pltpu.VMEM((N_IDXC, DIM, IDXC), jnp.int32),
          pltpu.VMEM((DIM, B_PER_W), jnp.float32),
          pltpu.VMEM((DIM, B_PER_W), jnp.float32),
          pltpu.VMEM((B_PER_W,), jnp.float32),
          pltpu.SemaphoreType.DMA,
      ],
  )
  return p2(uid, cid, ubm.reshape(-1), ibm.reshape(-1))
